# baseline (speedup 1.0000x reference)
import jax
import jax.numpy as jnp
from jax import lax
from jax.experimental import pallas as pl
from jax.experimental.pallas import tpu as pltpu
from jax.experimental.pallas import tpu_sc as plsc

B = 16384
D = 64
NC = 2
NS = 16
NW = NC * NS
BPW = B // NW
CHUNK = 64
NCHUNK = BPW // CHUNK

V_USER = 1000000
V_ITEM = 100000
V_CAT = 1000
V_BRAND = 1000

_PARAMS = pltpu.CompilerParams(use_tc_tiling_on_sc=True,
                               needs_layout_passes=False)
_MESH = plsc.VectorSubcoreMesh(core_axis_name="c", subcore_axis_name="s")


def _gat_body(u_id, i_id, c_id, b_id, us, is_, cs, bs, out_hbm,
              idx_raw, gidx_v, pb_v,
              b00, b01, b02, b03, b10, b11, b12, b13,
              ob0, ob1, gsem, wsem):
    bufs = ((b00, b01, b02, b03), (b10, b11, b12, b13))
    obuf = (ob0, ob1)
    wid = lax.axis_index("s") * NC + lax.axis_index("c")
    base = wid * BPW
    ids_hbm = (u_id, i_id, c_id, b_id)
    tabs = (us, is_, cs, bs)

    for f in range(4):
        pltpu.sync_copy(ids_hbm[f].at[pl.ds(base, BPW)], idx_raw.at[f])
    for f in range(4):
        for i in range(BPW // 16):
            v = idx_raw[f, pl.ds(i * 16, 16)]
            ci, lane = (i * 16) // CHUNK, (i * 16) % CHUNK
            gidx_v[f, ci, pl.ds(lane, 16)] = lax.shift_right_logical(v, 1)
            pb_v[f, ci, pl.ds(lane, 16)] = lax.shift_left(
                lax.bitwise_and(v, jnp.int32(1)), 6)

    def fire(ci, s):
        return [
            pltpu.async_copy(
                tabs[f].at[gidx_v.at[f, ci]], bufs[s][f], gsem)
            for f in range(4)
        ]

    def extract(ci, s):
        for f in range(4):
            src = bufs[s][f]
            dst = obuf[s]
            ocb = (f % 2) * 64

            def gloop(g, _):
                jv = lax.iota(jnp.int32, 16) + g * 16
                pb = pb_v[f, ci, pl.ds(g * 16, 16)]
                orow = 2 * jv + (1 if f >= 2 else 0)

                def wloop(wo, _2):
                    for wi in range(8):
                        wv = jnp.full((16,), wo * 8 + wi, dtype=jnp.int32)
                        vals = plsc.load_gather(src, [jv, pb + wv])
                        plsc.store_scatter(dst, [orow, wv + ocb], vals)
                    return _2
                lax.fori_loop(0, D // 8, wloop, 0)
                return _
            lax.fori_loop(0, CHUNK // 16, gloop, 0)

    g = [None] * NCHUNK
    w = [None] * NCHUNK
    g[0] = fire(0, 0)
    for ci in range(NCHUNK):
        s = ci % 2
        if ci + 1 < NCHUNK:
            g[ci + 1] = fire(ci + 1, 1 - s)
        for c in g[ci]:
            c.wait()
        if ci >= 2:
            w[ci - 2].wait()
        extract(ci, s)
        cbase = (base + ci * CHUNK) * 2
        w[ci] = pltpu.async_copy(
            obuf[s], out_hbm.at[pl.ds(cbase, 2 * CHUNK)], wsem)
    w[NCHUNK - 2].wait()
    w[NCHUNK - 1].wait()


_gat_call = pl.kernel(
    _gat_body,
    out_type=jax.ShapeDtypeStruct((2 * B, 128), jnp.float32),
    mesh=_MESH,
    compiler_params=_PARAMS,
    scratch_types=[
        pltpu.VMEM((4, BPW), jnp.int32),
        pltpu.VMEM((4, NCHUNK, CHUNK), jnp.int32),
        pltpu.VMEM((4, NCHUNK, CHUNK), jnp.int32),
    ]
    + [pltpu.VMEM((CHUNK, 128), jnp.float32) for _ in range(8)]
    + [
        pltpu.VMEM((2 * CHUNK, 128), jnp.float32),
        pltpu.VMEM((2 * CHUNK, 128), jnp.float32),
        pltpu.SemaphoreType.DMA,
        pltpu.SemaphoreType.DMA,
    ],
)


def _pack(t):
    return jnp.concatenate([t[0::2], t[1::2]], axis=1)


def kernel(user_id, item_id, category, brand,
           user_table, item_table, category_table, brand_table):
    us = _pack(user_table)
    is_ = _pack(item_table)
    cs = _pack(category_table)
    bs = _pack(brand_table)
    out = _gat_call(user_id, item_id, category, brand, us, is_, cs, bs)
    return out.reshape(B, 4 * D)


# speedup vs baseline: 9.7581x; 9.7581x over previous
import jax
import jax.numpy as jnp
from jax import lax
from jax.experimental import pallas as pl
from jax.experimental.pallas import tpu as pltpu
from jax.experimental.pallas import tpu_sc as plsc

B = 16384
D = 64
NC = 2
NS = 16
NW = NC * NS
BPW = B // NW
CHUNK = 64
NCHUNK = BPW // CHUNK

V_USER = 1000000
V_ITEM = 100000
V_CAT = 1000
V_BRAND = 1000

_PARAMS = pltpu.CompilerParams(use_tc_tiling_on_sc=True,
                               needs_layout_passes=False)
_MESH = plsc.VectorSubcoreMesh(core_axis_name="c", subcore_axis_name="s")


def _gat_body(u_id, i_id, c_id, b_id, us, is_, cs, bs, out_hbm,
              idx_raw, gidx_v, pb_v,
              b00, b01, b02, b03, b10, b11, b12, b13,
              ob0, ob1, gsem0, gsem1, wsem0, wsem1):
    gsems = (gsem0, gsem1)
    wsems = (wsem0, wsem1)
    bufs = ((b00, b01, b02, b03), (b10, b11, b12, b13))
    obuf = (ob0, ob1)
    wid = lax.axis_index("s") * NC + lax.axis_index("c")
    base = wid * BPW
    ids_hbm = (u_id, i_id, c_id, b_id)
    tabs = (us, is_, cs, bs)

    halves = (V_USER // 2, V_ITEM // 2, V_CAT // 2, V_BRAND // 2)
    for f in range(4):
        pltpu.sync_copy(ids_hbm[f].at[pl.ds(base, BPW)], idx_raw.at[f])
    for f in range(4):
        h = jnp.int32(halves[f])
        for i in range(BPW // 16):
            v = idx_raw[f, pl.ds(i * 16, 16)]
            ci, lane = (i * 16) // CHUNK, (i * 16) % CHUNK
            hi = jnp.where(v >= h, jnp.int32(1), jnp.int32(0))
            gidx_v[f, ci, pl.ds(lane, 16)] = v - hi * h
            pb_v[f, ci, pl.ds(lane, 16)] = hi * 64

    def fire(ci, s):
        return [
            pltpu.async_copy(
                tabs[f].at[gidx_v.at[f, ci]], bufs[s][f], gsems[s])
            for f in range(4)
        ]

    def extract(ci, s):
        for f in range(4):
            src = bufs[s][f]
            dst = obuf[s]
            ocb = (f % 2) * 64

            def gloop(g, _):
                jv = lax.iota(jnp.int32, 16) + g * 16
                pb = pb_v[f, ci, pl.ds(g * 16, 16)]
                orow = 2 * jv + (1 if f >= 2 else 0)

                def wloop(wo, _2):
                    for wi in range(8):
                        wv = jnp.full((16,), wo * 8 + wi, dtype=jnp.int32)
                        vals = plsc.load_gather(src, [jv, pb + wv])
                        plsc.store_scatter(dst, [orow, wv + ocb], vals)
                    return _2
                lax.fori_loop(0, D // 8, wloop, 0)
                return _
            lax.fori_loop(0, CHUNK // 16, gloop, 0)

    g = [None] * NCHUNK
    w = [None] * NCHUNK
    g[0] = fire(0, 0)
    for ci in range(NCHUNK):
        s = ci % 2
        if ci + 1 < NCHUNK:
            g[ci + 1] = fire(ci + 1, 1 - s)
        for c in g[ci]:
            c.wait()
        if ci >= 2:
            w[ci - 2].wait()
        extract(ci, s)
        cbase = (base + ci * CHUNK) * 2
        w[ci] = pltpu.async_copy(
            obuf[s], out_hbm.at[pl.ds(cbase, 2 * CHUNK)], wsems[s])
    w[NCHUNK - 2].wait()
    w[NCHUNK - 1].wait()


_gat_call = pl.kernel(
    _gat_body,
    out_type=jax.ShapeDtypeStruct((2 * B, 128), jnp.float32),
    mesh=_MESH,
    compiler_params=_PARAMS,
    scratch_types=[
        pltpu.VMEM((4, BPW), jnp.int32),
        pltpu.VMEM((4, NCHUNK, CHUNK), jnp.int32),
        pltpu.VMEM((4, NCHUNK, CHUNK), jnp.int32),
    ]
    + [pltpu.VMEM((CHUNK, 128), jnp.float32) for _ in range(8)]
    + [
        pltpu.VMEM((2 * CHUNK, 128), jnp.float32),
        pltpu.VMEM((2 * CHUNK, 128), jnp.float32),
        pltpu.SemaphoreType.DMA,
        pltpu.SemaphoreType.DMA,
        pltpu.SemaphoreType.DMA,
        pltpu.SemaphoreType.DMA,
    ],
)


def _pack(t):
    v = t.shape[0]
    return jnp.concatenate([t[: v // 2], t[v // 2:]], axis=1)


def kernel(user_id, item_id, category, brand,
           user_table, item_table, category_table, brand_table):
    us = _pack(user_table)
    is_ = _pack(item_table)
    cs = _pack(category_table)
    bs = _pack(brand_table)
    out = _gat_call(user_id, item_id, category, brand, us, is_, cs, bs)
    return out.reshape(B, 4 * D)


# speedup vs baseline: 13.4846x; 1.3819x over previous
import functools

import jax
import jax.numpy as jnp
from jax import lax
from jax.experimental import pallas as pl
from jax.experimental.pallas import tpu as pltpu
from jax.experimental.pallas import tpu_sc as plsc

B = 16384
D = 64
NC = 2
NS = 16
NW = NC * NS
BPW = B // NW
CHUNK = 128
NCHUNK = BPW // CHUNK
NSLOT = 3


def _body(uid, iid, cid, bid, ut, it, ct, bt, out_hbm,
          idx_v, rows_v, gsem0, gsem1, gsem2, wsem0, wsem1, wsem2):
    wid = lax.axis_index("s") * NC + lax.axis_index("c")
    idx_hbms = (uid, iid, cid, bid)
    tables = (ut, it, ct, bt)
    gsems = (gsem0, gsem1, gsem2)
    wsems = (wsem0, wsem1, wsem2)

    for f in range(4):
        pltpu.sync_copy(idx_hbms[f].at[wid], idx_v.at[f])

    def fire(ci, slot):
        return [
            pltpu.async_copy(
                tables[f].at[idx_v.at[f, ci]],
                rows_v.at[slot, f],
                gsems[slot],
            )
            for f in range(4)
        ]

    def drain_and_write(ci, slot, gcopies):
        cbase = wid * BPW + ci * CHUNK
        ws = []
        for f in range(4):
            gcopies[f].wait()
            ws.append(
                pltpu.async_copy(
                    rows_v.at[slot, f],
                    out_hbm.at[pl.ds(cbase, CHUNK), f],
                    wsems[slot],
                )
            )
        return ws

    g = [fire(0, 0), fire(1, 1), fire(2, 2), None]
    w = [None] * NCHUNK
    w[0] = drain_and_write(0, 0, g[0])
    w[1] = drain_and_write(1, 1, g[1])
    for c in w[0]:
        c.wait()
    g[3] = fire(3, 0)
    w[2] = drain_and_write(2, 2, g[2])
    w[3] = drain_and_write(3, 0, g[3])
    for ci in range(1, NCHUNK):
        for c in w[ci]:
            c.wait()


_sc_call = pl.kernel(
    _body,
    out_type=jax.ShapeDtypeStruct((B, 4, D), jnp.float32),
    mesh=plsc.VectorSubcoreMesh(core_axis_name="c", subcore_axis_name="s"),
    compiler_params=pltpu.CompilerParams(use_tc_tiling_on_sc=False),
    scratch_types=(
        [
            pltpu.VMEM((4, NCHUNK, CHUNK), jnp.int32),
            pltpu.VMEM((NSLOT, 4, CHUNK, D), jnp.float32),
        ]
        + [pltpu.SemaphoreType.DMA for _ in range(6)]
    ),
)


def kernel(user_id, item_id, category, brand,
           user_table, item_table, category_table, brand_table):
    shp = (NW, NCHUNK, CHUNK)
    out = _sc_call(
        user_id.reshape(shp), item_id.reshape(shp),
        category.reshape(shp), brand.reshape(shp),
        user_table, item_table, category_table, brand_table,
    )
    return out.reshape(B, 4 * D)
